# Initial kernel scaffold; baseline (speedup 1.0000x reference)
import jax
import jax.numpy as jnp
from jax.experimental import pallas as pl


def kernel(x, edge_index, W1_0, b1_0, W2_0, b2_0, gamma_0, beta_0, W1_1, b1_1, W2_1, b2_1, gamma_1, beta_1, W1_2, b1_2, W2_2, b2_2, gamma_2, beta_2):
    raise NotImplementedError("write your pallas kernel here")


# speedup vs baseline: 4.4715x; 4.4715x over previous
import functools

import jax
import jax.numpy as jnp
from jax import lax
from jax.experimental import pallas as pl
from jax.experimental.pallas import tpu as pltpu
from jax.experimental.pallas import tpu_sc as plsc

N = 10000
E = 320000
D = 128
BN_EPS = 1e-5

NC = 2
NS = 16
NW = NC * NS
EPW = E // NW
C = 80
NCHUNK = EPW // C
NPAD = 10240
RPT = NPAD // NS


def _agg_body(h_hbm, src_hbm, dst_hbm, out_hbm, src_v, dst_v, rows_v, agg_sh, sem):
    c = lax.axis_index("c")
    s = lax.axis_index("s")

    def zbody(i, _):
        r = i // 8
        col = (i % 8) * 16
        rows_v[r, pl.ds(col, 16)] = jnp.zeros((16,), jnp.float32)
        return 0
    lax.fori_loop(0, C * 8, zbody, 0)
    base_r = s * RPT

    def zcopy(j, _):
        pltpu.sync_copy(rows_v, agg_sh.at[pl.ds(base_r + j * C, C)])
        return 0
    lax.fori_loop(0, RPT // C, zcopy, 0)
    plsc.subcore_barrier()

    w = c * NS + s
    base_e = w * EPW

    def body(k, _):
        off = base_e + k * C
        pltpu.sync_copy(src_hbm.at[pl.ds(off, C)], src_v)
        pltpu.sync_copy(dst_hbm.at[pl.ds(off, C)], dst_v)
        pltpu.async_copy(h_hbm.at[src_v], rows_v, sem).wait()
        pltpu.sync_copy(rows_v, agg_sh.at[dst_v], add=True)
        return 0
    lax.fori_loop(0, NCHUNK, body, 0)
    plsc.subcore_barrier()

    out_base = c * NPAD + s * RPT
    pltpu.sync_copy(agg_sh.at[pl.ds(base_r, RPT)], out_hbm.at[pl.ds(out_base, RPT)])


_agg_call_cache = []


def _get_agg_call():
    if not _agg_call_cache:
        _agg_call_cache.append(functools.partial(
            pl.kernel,
            mesh=plsc.VectorSubcoreMesh(core_axis_name="c", subcore_axis_name="s"),
            out_type=jax.ShapeDtypeStruct((2 * NPAD, D), jnp.float32),
            scratch_types=[
                pltpu.VMEM((C,), jnp.int32),
                pltpu.VMEM((C,), jnp.int32),
                pltpu.VMEM((C, D), jnp.float32),
                pltpu.VMEM_SHARED((NPAD, D), jnp.float32),
                pltpu.SemaphoreType.DMA,
            ],
        )(_agg_body))
    return _agg_call_cache[0]


_BLK = 1000
_GRID = N // _BLK


def _mlp_body(h_ref, agg_ref, w1_ref, b1_ref, w2_ref, b2_ref, o_ref, sums_ref):
    i = pl.program_id(0)
    x = h_ref[...] + agg_ref[0] + agg_ref[1]
    t = jnp.maximum(
        jnp.dot(x, w1_ref[...], preferred_element_type=jnp.float32) + b1_ref[...], 0.0)
    o = jnp.dot(t, w2_ref[...], preferred_element_type=jnp.float32) + b2_ref[...]
    o_ref[...] = o
    st = jnp.concatenate([jnp.sum(o, 0, keepdims=True),
                          jnp.sum(o * o, 0, keepdims=True)], axis=0)

    @pl.when(i == 0)
    def _():
        sums_ref[...] = st

    @pl.when(i > 0)
    def _():
        sums_ref[...] += st


_mlp_call = pl.pallas_call(
    _mlp_body,
    grid=(_GRID,),
    in_specs=[
        pl.BlockSpec((_BLK, D), lambda i: (i, 0)),
        pl.BlockSpec((2, _BLK, D), lambda i: (0, i, 0)),
        pl.BlockSpec((D, D), lambda i: (0, 0)),
        pl.BlockSpec((1, D), lambda i: (0, 0)),
        pl.BlockSpec((D, D), lambda i: (0, 0)),
        pl.BlockSpec((1, D), lambda i: (0, 0)),
    ],
    out_specs=[
        pl.BlockSpec((_BLK, D), lambda i: (i, 0)),
        pl.BlockSpec((2, D), lambda i: (0, 0)),
    ],
    out_shape=[
        jax.ShapeDtypeStruct((N, D), jnp.float32),
        jax.ShapeDtypeStruct((2, D), jnp.float32),
    ],
)


def _bn_body(o_ref, sums_ref, g_ref, bt_ref, out_ref):
    inv_n = 1.0 / N
    mean = sums_ref[pl.ds(0, 1), :] * inv_n
    ex2 = sums_ref[pl.ds(1, 1), :] * inv_n
    var = ex2 - mean * mean
    scale = g_ref[...] * lax.rsqrt(var + BN_EPS)
    out_ref[...] = jnp.maximum((o_ref[...] - mean) * scale + bt_ref[...], 0.0)


_bn_call = pl.pallas_call(
    _bn_body,
    grid=(_GRID,),
    in_specs=[
        pl.BlockSpec((_BLK, D), lambda i: (i, 0)),
        pl.BlockSpec((2, D), lambda i: (0, 0)),
        pl.BlockSpec((1, D), lambda i: (0, 0)),
        pl.BlockSpec((1, D), lambda i: (0, 0)),
    ],
    out_specs=pl.BlockSpec((_BLK, D), lambda i: (i, 0)),
    out_shape=jax.ShapeDtypeStruct((N, D), jnp.float32),
)


def kernel(x, edge_index,
           W1_0, b1_0, W2_0, b2_0, gamma_0, beta_0,
           W1_1, b1_1, W2_1, b2_1, gamma_1, beta_1,
           W1_2, b1_2, W2_2, b2_2, gamma_2, beta_2):
    src = edge_index[0]
    dst = edge_index[1]
    layer_params = [
        (W1_0, b1_0, W2_0, b2_0, gamma_0, beta_0),
        (W1_1, b1_1, W2_1, b2_1, gamma_1, beta_1),
        (W1_2, b1_2, W2_2, b2_2, gamma_2, beta_2),
    ]
    h = x
    outs = []
    for (W1, b1, W2, b2, g, bt) in layer_params:
        aggf = _get_agg_call()(h, src, dst)
        agg2 = aggf.reshape(2, NPAD, D)
        o, sums = _mlp_call(h, agg2, W1, b1.reshape(1, D), W2, b2.reshape(1, D))
        h = _bn_call(o, sums, g.reshape(1, D), bt.reshape(1, D))
        outs.append(h)
    return tuple(outs)


# speedup vs baseline: 11.1568x; 2.4951x over previous
import functools

import jax
import jax.numpy as jnp
from jax import lax
from jax.experimental import pallas as pl
from jax.experimental.pallas import tpu as pltpu
from jax.experimental.pallas import tpu_sc as plsc

N = 10000
E = 320000
D = 128
BN_EPS = 1e-5

NC = 2
NS = 16
NW = NC * NS
EPW = E // NW
C = 40
NCHUNK = EPW // C
NPAD = 10240
RPT = NPAD // NS


G = 5


def _agg_body(h_hbm, src_hbm, dst_hbm, out_hbm, src_i, dst_i, ring_v, agg_sh,
              *sems):
    sem_i = sems[0:G]
    sem_g = sems[G:2 * G]
    sem_s = sems[2 * G:3 * G]
    c = lax.axis_index("c")
    s = lax.axis_index("s")
    w = c * NS + s
    base_e = w * EPW

    def zbody(i, _):
        r = i // 8
        col = (i % 8) * 16
        ring_v[0, r, pl.ds(col, 16)] = jnp.zeros((16,), jnp.float32)
        return 0
    lax.fori_loop(0, C * 8, zbody, 0)
    base_r = s * RPT

    def zcopy(j, _):
        pltpu.sync_copy(ring_v.at[0], agg_sh.at[pl.ds(base_r + j * C, C)])
        return 0
    lax.fori_loop(0, RPT // C, zcopy, 0)
    plsc.subcore_barrier()

    def idx_issue(t, b):
        off = base_e + t * C
        pltpu.async_copy(src_hbm.at[pl.ds(off, C)], src_i.at[b], sem_i[b])
        pltpu.async_copy(dst_hbm.at[pl.ds(off, C)], dst_i.at[b], sem_i[b])

    def idx_wait(t, b):
        off = base_e + t * C
        pltpu.make_async_copy(src_hbm.at[pl.ds(off, C)], src_i.at[b], sem_i[b]).wait()
        pltpu.make_async_copy(dst_hbm.at[pl.ds(off, C)], dst_i.at[b], sem_i[b]).wait()

    def gather_issue(b):
        pltpu.async_copy(h_hbm.at[src_i.at[b]], ring_v.at[b], sem_g[b])

    def gather_wait(b):
        pltpu.make_async_copy(h_hbm.at[src_i.at[b]], ring_v.at[b], sem_g[b]).wait()

    def scat_issue(b):
        pltpu.async_copy(ring_v.at[b], agg_sh.at[dst_i.at[b]], sem_s[b], add=True)

    def scat_wait(b):
        pltpu.make_async_copy(ring_v.at[b], agg_sh.at[dst_i.at[b]], sem_s[b]).wait()

    for t in range(4):
        idx_issue(t, t)
    for t in range(2):
        idx_wait(t, t)
        gather_issue(t)

    def body(j, _):
        for b in range(G):
            t = j * G + b

            @pl.when(t + 4 < NCHUNK)
            def _():
                idx_issue(t + 4, (b + 4) % G)

            @pl.when(t + 2 < NCHUNK)
            def _():
                idx_wait(t + 2, (b + 2) % G)
                gather_issue((b + 2) % G)
            gather_wait(b)
            scat_issue(b)

            @pl.when(t >= 2)
            def _():
                scat_wait((b - 2) % G)
        return 0
    lax.fori_loop(0, NCHUNK // G, body, 0)
    scat_wait((NCHUNK - 2) % G)
    scat_wait((NCHUNK - 1) % G)
    plsc.subcore_barrier()

    out_base = c * NPAD + s * RPT
    pltpu.sync_copy(agg_sh.at[pl.ds(base_r, RPT)], out_hbm.at[pl.ds(out_base, RPT)])


_agg_call_cache = []


def _get_agg_call():
    if not _agg_call_cache:
        _agg_call_cache.append(functools.partial(
            pl.kernel,
            mesh=plsc.VectorSubcoreMesh(core_axis_name="c", subcore_axis_name="s"),
            out_type=jax.ShapeDtypeStruct((2 * NPAD, D), jnp.float32),
            scratch_types=[
                pltpu.VMEM((G, C), jnp.int32),
                pltpu.VMEM((G, C), jnp.int32),
                pltpu.VMEM((G, C, D), jnp.float32),
                pltpu.VMEM_SHARED((NPAD, D), jnp.float32),
            ] + [pltpu.SemaphoreType.DMA] * (3 * G),
        )(_agg_body))
    return _agg_call_cache[0]


_BLK = 1000
_GRID = N // _BLK


def _mlp_body(h_ref, agg_ref, w1_ref, b1_ref, w2_ref, b2_ref, o_ref, sums_ref):
    i = pl.program_id(0)
    x = h_ref[...] + agg_ref[0] + agg_ref[1]
    t = jnp.maximum(
        jnp.dot(x, w1_ref[...], preferred_element_type=jnp.float32) + b1_ref[...], 0.0)
    o = jnp.dot(t, w2_ref[...], preferred_element_type=jnp.float32) + b2_ref[...]
    o_ref[...] = o
    st = jnp.concatenate([jnp.sum(o, 0, keepdims=True),
                          jnp.sum(o * o, 0, keepdims=True)], axis=0)

    @pl.when(i == 0)
    def _():
        sums_ref[...] = st

    @pl.when(i > 0)
    def _():
        sums_ref[...] += st


_mlp_call = pl.pallas_call(
    _mlp_body,
    grid=(_GRID,),
    in_specs=[
        pl.BlockSpec((_BLK, D), lambda i: (i, 0)),
        pl.BlockSpec((2, _BLK, D), lambda i: (0, i, 0)),
        pl.BlockSpec((D, D), lambda i: (0, 0)),
        pl.BlockSpec((1, D), lambda i: (0, 0)),
        pl.BlockSpec((D, D), lambda i: (0, 0)),
        pl.BlockSpec((1, D), lambda i: (0, 0)),
    ],
    out_specs=[
        pl.BlockSpec((_BLK, D), lambda i: (i, 0)),
        pl.BlockSpec((2, D), lambda i: (0, 0)),
    ],
    out_shape=[
        jax.ShapeDtypeStruct((N, D), jnp.float32),
        jax.ShapeDtypeStruct((2, D), jnp.float32),
    ],
)


def _bn_body(o_ref, sums_ref, g_ref, bt_ref, out_ref):
    inv_n = 1.0 / N
    mean = sums_ref[pl.ds(0, 1), :] * inv_n
    ex2 = sums_ref[pl.ds(1, 1), :] * inv_n
    var = ex2 - mean * mean
    scale = g_ref[...] * lax.rsqrt(var + BN_EPS)
    out_ref[...] = jnp.maximum((o_ref[...] - mean) * scale + bt_ref[...], 0.0)


_bn_call = pl.pallas_call(
    _bn_body,
    grid=(_GRID,),
    in_specs=[
        pl.BlockSpec((_BLK, D), lambda i: (i, 0)),
        pl.BlockSpec((2, D), lambda i: (0, 0)),
        pl.BlockSpec((1, D), lambda i: (0, 0)),
        pl.BlockSpec((1, D), lambda i: (0, 0)),
    ],
    out_specs=pl.BlockSpec((_BLK, D), lambda i: (i, 0)),
    out_shape=jax.ShapeDtypeStruct((N, D), jnp.float32),
)


def kernel(x, edge_index,
           W1_0, b1_0, W2_0, b2_0, gamma_0, beta_0,
           W1_1, b1_1, W2_1, b2_1, gamma_1, beta_1,
           W1_2, b1_2, W2_2, b2_2, gamma_2, beta_2):
    src = edge_index[0]
    dst = edge_index[1]
    layer_params = [
        (W1_0, b1_0, W2_0, b2_0, gamma_0, beta_0),
        (W1_1, b1_1, W2_1, b2_1, gamma_1, beta_1),
        (W1_2, b1_2, W2_2, b2_2, gamma_2, beta_2),
    ]
    h = x
    outs = []
    for (W1, b1, W2, b2, g, bt) in layer_params:
        aggf = _get_agg_call()(h, src, dst)
        agg2 = aggf.reshape(2, NPAD, D)
        o, sums = _mlp_call(h, agg2, W1, b1.reshape(1, D), W2, b2.reshape(1, D))
        h = _bn_call(o, sums, g.reshape(1, D), bt.reshape(1, D))
        outs.append(h)
    return tuple(outs)


# speedup vs baseline: 11.6382x; 1.0431x over previous
import functools

import jax
import jax.numpy as jnp
from jax import lax
from jax.experimental import pallas as pl
from jax.experimental.pallas import tpu as pltpu
from jax.experimental.pallas import tpu_sc as plsc

N = 10000
E = 320000
D = 128
BN_EPS = 1e-5

NC = 2
NS = 16
NW = NC * NS
EPW = E // NW
C = 40
NCHUNK = EPW // C
NPAD = 10240
RPT = NPAD // NS


G = 5


def _agg_body(h_hbm, src_hbm, dst_hbm, out_hbm, src_i, dst_i, ring_v, agg_sh,
              *sems):
    sem_i = sems[0:G]
    sem_g = sems[G:2 * G]
    sem_s = sems[2 * G:3 * G]
    c = lax.axis_index("c")
    s = lax.axis_index("s")
    w = c * NS + s
    base_e = w * EPW

    def zbody(i, _):
        r = i // 8
        col = (i % 8) * 16
        ring_v[0, r, pl.ds(col, 16)] = jnp.zeros((16,), jnp.float32)
        return 0
    lax.fori_loop(0, C * 8, zbody, 0)
    base_r = s * RPT

    def zcopy(j, _):
        pltpu.sync_copy(ring_v.at[0], agg_sh.at[pl.ds(base_r + j * C, C)])
        return 0
    lax.fori_loop(0, RPT // C, zcopy, 0)
    plsc.subcore_barrier()

    def idx_issue(t, b):
        off = base_e + t * C
        pltpu.async_copy(src_hbm.at[pl.ds(off, C)], src_i.at[b], sem_i[b])
        pltpu.async_copy(dst_hbm.at[pl.ds(off, C)], dst_i.at[b], sem_i[b])

    def idx_wait(t, b):
        off = base_e + t * C
        pltpu.make_async_copy(src_hbm.at[pl.ds(off, C)], src_i.at[b], sem_i[b]).wait()
        pltpu.make_async_copy(dst_hbm.at[pl.ds(off, C)], dst_i.at[b], sem_i[b]).wait()

    def gather_issue(b):
        pltpu.async_copy(h_hbm.at[src_i.at[b]], ring_v.at[b], sem_g[b])

    def gather_wait(b):
        pltpu.make_async_copy(h_hbm.at[src_i.at[b]], ring_v.at[b], sem_g[b]).wait()

    def scat_issue(b):
        pltpu.async_copy(ring_v.at[b], agg_sh.at[dst_i.at[b]], sem_s[b], add=True)

    def scat_wait(b):
        pltpu.make_async_copy(ring_v.at[b], agg_sh.at[dst_i.at[b]], sem_s[b]).wait()

    for t in range(4):
        idx_issue(t, t)
    for t in range(2):
        idx_wait(t, t)
        gather_issue(t)

    def body(j, _):
        for b in range(G):
            t = j * G + b

            @pl.when(t + 4 < NCHUNK)
            def _():
                idx_issue(t + 4, (b + 4) % G)

            @pl.when(t + 2 < NCHUNK)
            def _():
                idx_wait(t + 2, (b + 2) % G)
                gather_issue((b + 2) % G)
            gather_wait(b)
            scat_issue(b)

            @pl.when(t >= 2)
            def _():
                scat_wait((b - 2) % G)
        return 0
    lax.fori_loop(0, NCHUNK // G, body, 0)
    scat_wait((NCHUNK - 2) % G)
    scat_wait((NCHUNK - 1) % G)
    plsc.subcore_barrier()

    out_base = c * NPAD + s * RPT
    pltpu.sync_copy(agg_sh.at[pl.ds(base_r, RPT)], out_hbm.at[pl.ds(out_base, RPT)])


_agg_call_cache = []


def _get_agg_call():
    if not _agg_call_cache:
        _agg_call_cache.append(functools.partial(
            pl.kernel,
            mesh=plsc.VectorSubcoreMesh(core_axis_name="c", subcore_axis_name="s"),
            out_type=jax.ShapeDtypeStruct((2 * NPAD, D), jnp.float32),
            scratch_types=[
                pltpu.VMEM((G, C), jnp.int32),
                pltpu.VMEM((G, C), jnp.int32),
                pltpu.VMEM((G, C, D), jnp.float32),
                pltpu.VMEM_SHARED((NPAD, D), jnp.float32),
            ] + [pltpu.SemaphoreType.DMA] * (3 * G),
        )(_agg_body))
    return _agg_call_cache[0]


_BLK = 1000
_GRID = N // _BLK


def _mlpbn_body(h_ref, agg_ref, w1_ref, b1_ref, w2_ref, b2_ref, g_ref, bt_ref,
                out_ref, o_scr, sums_scr):
    i = pl.program_id(0)

    @pl.when(i < _GRID)
    def _():
        x = h_ref[...] + agg_ref[0] + agg_ref[1]
        t = jnp.maximum(
            jnp.dot(x.astype(jnp.bfloat16), w1_ref[...].astype(jnp.bfloat16),
                    preferred_element_type=jnp.float32) + b1_ref[...], 0.0)
        o = jnp.dot(t.astype(jnp.bfloat16), w2_ref[...].astype(jnp.bfloat16),
                    preferred_element_type=jnp.float32) + b2_ref[...]
        o_scr[pl.ds(i * _BLK, _BLK), :] = o
        st = jnp.concatenate([jnp.sum(o, 0, keepdims=True),
                              jnp.sum(o * o, 0, keepdims=True)], axis=0)

        @pl.when(i == 0)
        def _():
            sums_scr[...] = st

        @pl.when(i > 0)
        def _():
            sums_scr[...] += st

    @pl.when(i >= _GRID)
    def _():
        j = i - _GRID
        inv_n = 1.0 / N
        mean = sums_scr[pl.ds(0, 1), :] * inv_n
        ex2 = sums_scr[pl.ds(1, 1), :] * inv_n
        var = ex2 - mean * mean
        scale = g_ref[...] * lax.rsqrt(var + BN_EPS)
        o = o_scr[pl.ds(j * _BLK, _BLK), :]
        out_ref[...] = jnp.maximum((o - mean) * scale + bt_ref[...], 0.0)


_mlpbn_call = pl.pallas_call(
    _mlpbn_body,
    grid=(2 * _GRID,),
    in_specs=[
        pl.BlockSpec((_BLK, D), lambda i: (jnp.minimum(i, _GRID - 1), 0)),
        pl.BlockSpec((2, _BLK, D), lambda i: (0, jnp.minimum(i, _GRID - 1), 0)),
        pl.BlockSpec((D, D), lambda i: (0, 0)),
        pl.BlockSpec((1, D), lambda i: (0, 0)),
        pl.BlockSpec((D, D), lambda i: (0, 0)),
        pl.BlockSpec((1, D), lambda i: (0, 0)),
        pl.BlockSpec((1, D), lambda i: (0, 0)),
        pl.BlockSpec((1, D), lambda i: (0, 0)),
    ],
    out_specs=pl.BlockSpec((_BLK, D), lambda i: (jnp.maximum(i - _GRID, 0), 0)),
    out_shape=jax.ShapeDtypeStruct((N, D), jnp.float32),
    scratch_shapes=[
        pltpu.VMEM((N, D), jnp.float32),
        pltpu.VMEM((2, D), jnp.float32),
    ],
)


def kernel(x, edge_index,
           W1_0, b1_0, W2_0, b2_0, gamma_0, beta_0,
           W1_1, b1_1, W2_1, b2_1, gamma_1, beta_1,
           W1_2, b1_2, W2_2, b2_2, gamma_2, beta_2):
    src = edge_index[0]
    dst = edge_index[1]
    layer_params = [
        (W1_0, b1_0, W2_0, b2_0, gamma_0, beta_0),
        (W1_1, b1_1, W2_1, b2_1, gamma_1, beta_1),
        (W1_2, b1_2, W2_2, b2_2, gamma_2, beta_2),
    ]
    h = x
    outs = []
    for (W1, b1, W2, b2, g, bt) in layer_params:
        aggf = _get_agg_call()(h, src, dst)
        agg2 = aggf.reshape(2, NPAD, D)
        h = _mlpbn_call(h, agg2, W1, b1.reshape(1, D), W2, b2.reshape(1, D),
                        g.reshape(1, D), bt.reshape(1, D))
        outs.append(h)
    return tuple(outs)


# speedup vs baseline: 11.6563x; 1.0016x over previous
import functools

import jax
import jax.numpy as jnp
from jax import lax
from jax.experimental import pallas as pl
from jax.experimental.pallas import tpu as pltpu
from jax.experimental.pallas import tpu_sc as plsc

N = 10000
E = 320000
D = 128
BN_EPS = 1e-5

NC = 2
NS = 16
NW = NC * NS
EPW = E // NW
C = 40
NCHUNK = EPW // C
NPAD = 10240
RPT = NPAD // NS


G = 5


def _agg_body(h_hbm, src_hbm, dst_hbm, out_hbm, src_i, dst_i, ring_v, agg_sh,
              *sems):
    sem_i = sems[0:G]
    sem_g = sems[G:2 * G]
    sem_s = sems[2 * G:3 * G]
    c = lax.axis_index("c")
    s = lax.axis_index("s")
    w = c * NS + s
    base_e = w * EPW

    def zbody(i, _):
        r = i // 8
        col = (i % 8) * 16
        ring_v[0, r, pl.ds(col, 16)] = jnp.zeros((16,), jnp.float32)
        return 0
    lax.fori_loop(0, C * 8, zbody, 0)
    base_r = s * RPT

    def zcopy(j, _):
        pltpu.sync_copy(ring_v.at[0], agg_sh.at[pl.ds(base_r + j * C, C)])
        return 0
    lax.fori_loop(0, RPT // C, zcopy, 0)
    plsc.subcore_barrier()

    def idx_issue(t, b):
        off = base_e + t * C
        pltpu.async_copy(src_hbm.at[pl.ds(off, C)], src_i.at[b], sem_i[b])
        pltpu.async_copy(dst_hbm.at[pl.ds(off, C)], dst_i.at[b], sem_i[b])

    def idx_wait(t, b):
        off = base_e + t * C
        pltpu.make_async_copy(src_hbm.at[pl.ds(off, C)], src_i.at[b], sem_i[b]).wait()
        pltpu.make_async_copy(dst_hbm.at[pl.ds(off, C)], dst_i.at[b], sem_i[b]).wait()

    def gather_issue(b):
        pltpu.async_copy(h_hbm.at[src_i.at[b]], ring_v.at[b], sem_g[b])

    def gather_wait(b):
        pltpu.make_async_copy(h_hbm.at[src_i.at[b]], ring_v.at[b], sem_g[b]).wait()

    def scat_issue(b):
        pltpu.async_copy(ring_v.at[b], agg_sh.at[dst_i.at[b]], sem_s[b], add=True)

    def scat_wait(b):
        pltpu.make_async_copy(ring_v.at[b], agg_sh.at[dst_i.at[b]], sem_s[b]).wait()

    for t in range(4):
        idx_issue(t, t)
    for t in range(2):
        idx_wait(t, t)
        gather_issue(t)

    def body(j, _):
        for b in range(G):
            t = j * G + b

            @pl.when(t + 4 < NCHUNK)
            def _():
                idx_issue(t + 4, (b + 4) % G)

            @pl.when(t >= 3)
            def _():
                scat_wait((b + 2) % G)

            @pl.when(t + 2 < NCHUNK)
            def _():
                idx_wait(t + 2, (b + 2) % G)
                gather_issue((b + 2) % G)
            gather_wait(b)
            scat_issue(b)
        return 0
    lax.fori_loop(0, NCHUNK // G, body, 0)
    scat_wait((NCHUNK - 3) % G)
    scat_wait((NCHUNK - 2) % G)
    scat_wait((NCHUNK - 1) % G)
    plsc.subcore_barrier()

    out_base = c * NPAD + s * RPT
    pltpu.sync_copy(agg_sh.at[pl.ds(base_r, RPT)], out_hbm.at[pl.ds(out_base, RPT)])


_agg_call_cache = []


def _get_agg_call():
    if not _agg_call_cache:
        _agg_call_cache.append(functools.partial(
            pl.kernel,
            mesh=plsc.VectorSubcoreMesh(core_axis_name="c", subcore_axis_name="s"),
            out_type=jax.ShapeDtypeStruct((2 * NPAD, D), jnp.float32),
            scratch_types=[
                pltpu.VMEM((G, C), jnp.int32),
                pltpu.VMEM((G, C), jnp.int32),
                pltpu.VMEM((G, C, D), jnp.float32),
                pltpu.VMEM_SHARED((NPAD, D), jnp.float32),
            ] + [pltpu.SemaphoreType.DMA] * (3 * G),
        )(_agg_body))
    return _agg_call_cache[0]


_BLK = 1000
_GRID = N // _BLK


def _mlpbn_body(h_ref, agg_ref, w1_ref, b1_ref, w2_ref, b2_ref, g_ref, bt_ref,
                out_ref, o_scr, sums_scr):
    i = pl.program_id(0)

    @pl.when(i < _GRID)
    def _():
        x = h_ref[...] + agg_ref[0] + agg_ref[1]
        t = jnp.maximum(
            jnp.dot(x.astype(jnp.bfloat16), w1_ref[...].astype(jnp.bfloat16),
                    preferred_element_type=jnp.float32) + b1_ref[...], 0.0)
        o = jnp.dot(t.astype(jnp.bfloat16), w2_ref[...].astype(jnp.bfloat16),
                    preferred_element_type=jnp.float32) + b2_ref[...]
        o_scr[pl.ds(i * _BLK, _BLK), :] = o
        st = jnp.concatenate([jnp.sum(o, 0, keepdims=True),
                              jnp.sum(o * o, 0, keepdims=True)], axis=0)

        @pl.when(i == 0)
        def _():
            sums_scr[...] = st

        @pl.when(i > 0)
        def _():
            sums_scr[...] += st

    @pl.when(i >= _GRID)
    def _():
        j = i - _GRID
        inv_n = 1.0 / N
        mean = sums_scr[pl.ds(0, 1), :] * inv_n
        ex2 = sums_scr[pl.ds(1, 1), :] * inv_n
        var = ex2 - mean * mean
        scale = g_ref[...] * lax.rsqrt(var + BN_EPS)
        o = o_scr[pl.ds(j * _BLK, _BLK), :]
        out_ref[...] = jnp.maximum((o - mean) * scale + bt_ref[...], 0.0)


_mlpbn_call = pl.pallas_call(
    _mlpbn_body,
    grid=(2 * _GRID,),
    in_specs=[
        pl.BlockSpec((_BLK, D), lambda i: (jnp.minimum(i, _GRID - 1), 0)),
        pl.BlockSpec((2, _BLK, D), lambda i: (0, jnp.minimum(i, _GRID - 1), 0)),
        pl.BlockSpec((D, D), lambda i: (0, 0)),
        pl.BlockSpec((1, D), lambda i: (0, 0)),
        pl.BlockSpec((D, D), lambda i: (0, 0)),
        pl.BlockSpec((1, D), lambda i: (0, 0)),
        pl.BlockSpec((1, D), lambda i: (0, 0)),
        pl.BlockSpec((1, D), lambda i: (0, 0)),
    ],
    out_specs=pl.BlockSpec((_BLK, D), lambda i: (jnp.maximum(i - _GRID, 0), 0)),
    out_shape=jax.ShapeDtypeStruct((N, D), jnp.float32),
    scratch_shapes=[
        pltpu.VMEM((N, D), jnp.float32),
        pltpu.VMEM((2, D), jnp.float32),
    ],
)


def kernel(x, edge_index,
           W1_0, b1_0, W2_0, b2_0, gamma_0, beta_0,
           W1_1, b1_1, W2_1, b2_1, gamma_1, beta_1,
           W1_2, b1_2, W2_2, b2_2, gamma_2, beta_2):
    src = edge_index[0]
    dst = edge_index[1]
    layer_params = [
        (W1_0, b1_0, W2_0, b2_0, gamma_0, beta_0),
        (W1_1, b1_1, W2_1, b2_1, gamma_1, beta_1),
        (W1_2, b1_2, W2_2, b2_2, gamma_2, beta_2),
    ]
    h = x
    outs = []
    for (W1, b1, W2, b2, g, bt) in layer_params:
        aggf = _get_agg_call()(h, src, dst)
        agg2 = aggf.reshape(2, NPAD, D)
        h = _mlpbn_call(h, agg2, W1, b1.reshape(1, D), W2, b2.reshape(1, D),
                        g.reshape(1, D), bt.reshape(1, D))
        outs.append(h)
    return tuple(outs)


# speedup vs baseline: 11.8066x; 1.0129x over previous
import functools

import jax
import jax.numpy as jnp
from jax import lax
from jax.experimental import pallas as pl
from jax.experimental.pallas import tpu as pltpu
from jax.experimental.pallas import tpu_sc as plsc

N = 10000
E = 320000
D = 128
BN_EPS = 1e-5

NC = 2
NS = 16
NW = NC * NS
EPW = E // NW
C = 48
EPP = 10080
NCHUNK = EPP // C
NPAD = 10240
RPT = NPAD // NS


G = 5


def _agg_body(h_hbm, src_hbm, dst_hbm, out_hbm, src_i, dst_i, ring_v, agg_sh,
              *sems):
    sem_i = sems[0:G]
    sem_g = sems[G:2 * G]
    sem_s = sems[2 * G:3 * G]
    c = lax.axis_index("c")
    s = lax.axis_index("s")
    w = c * NS + s
    base_e = w * EPP

    def zbody(i, _):
        r = i // 8
        col = (i % 8) * 16
        ring_v[0, r, pl.ds(col, 16)] = jnp.zeros((16,), jnp.float32)
        return 0
    lax.fori_loop(0, C * 8, zbody, 0)
    base_r = s * RPT

    ZC = 32

    def zcopy(j, _):
        pltpu.sync_copy(ring_v.at[0, pl.ds(0, ZC)],
                        agg_sh.at[pl.ds(base_r + j * ZC, ZC)])
        return 0
    lax.fori_loop(0, RPT // ZC, zcopy, 0)
    plsc.subcore_barrier()

    def idx_issue(t, b):
        off = base_e + t * C
        pltpu.async_copy(src_hbm.at[pl.ds(off, C)], src_i.at[b], sem_i[b])
        pltpu.async_copy(dst_hbm.at[pl.ds(off, C)], dst_i.at[b], sem_i[b])

    def idx_wait(t, b):
        off = base_e + t * C
        pltpu.make_async_copy(src_hbm.at[pl.ds(off, C)], src_i.at[b], sem_i[b]).wait()
        pltpu.make_async_copy(dst_hbm.at[pl.ds(off, C)], dst_i.at[b], sem_i[b]).wait()

    def gather_issue(b):
        pltpu.async_copy(h_hbm.at[src_i.at[b]], ring_v.at[b], sem_g[b])

    def gather_wait(b):
        pltpu.make_async_copy(h_hbm.at[src_i.at[b]], ring_v.at[b], sem_g[b]).wait()

    def scat_issue(b):
        pltpu.async_copy(ring_v.at[b], agg_sh.at[dst_i.at[b]], sem_s[b], add=True)

    def scat_wait(b):
        pltpu.make_async_copy(ring_v.at[b], agg_sh.at[dst_i.at[b]], sem_s[b]).wait()

    for t in range(4):
        idx_issue(t, t)
    for t in range(2):
        idx_wait(t, t)
        gather_issue(t)

    def body(j, _):
        for b in range(G):
            t = j * G + b

            @pl.when(t + 4 < NCHUNK)
            def _():
                idx_issue(t + 4, (b + 4) % G)

            @pl.when(t >= 3)
            def _():
                scat_wait((b + 2) % G)

            @pl.when(t + 2 < NCHUNK)
            def _():
                idx_wait(t + 2, (b + 2) % G)
                gather_issue((b + 2) % G)
            gather_wait(b)
            scat_issue(b)
        return 0
    lax.fori_loop(0, NCHUNK // G, body, 0)
    scat_wait((NCHUNK - 3) % G)
    scat_wait((NCHUNK - 2) % G)
    scat_wait((NCHUNK - 1) % G)
    plsc.subcore_barrier()

    out_base = c * NPAD + s * RPT
    pltpu.sync_copy(agg_sh.at[pl.ds(base_r, RPT)], out_hbm.at[pl.ds(out_base, RPT)])


_agg_call_cache = []


def _get_agg_call():
    if not _agg_call_cache:
        _agg_call_cache.append(functools.partial(
            pl.kernel,
            mesh=plsc.VectorSubcoreMesh(core_axis_name="c", subcore_axis_name="s"),
            out_type=jax.ShapeDtypeStruct((2 * NPAD, D), jnp.float32),
            scratch_types=[
                pltpu.VMEM((G, C), jnp.int32),
                pltpu.VMEM((G, C), jnp.int32),
                pltpu.VMEM((G, C, D), jnp.float32),
                pltpu.VMEM_SHARED((NPAD, D), jnp.float32),
            ] + [pltpu.SemaphoreType.DMA] * (3 * G),
        )(_agg_body))
    return _agg_call_cache[0]


_BLK = 1000
_GRID = N // _BLK


def _mlpbn_body(h_ref, agg_ref, w1_ref, b1_ref, w2_ref, b2_ref, g_ref, bt_ref,
                out_ref, o_scr, sums_scr):
    i = pl.program_id(0)

    @pl.when(i < _GRID)
    def _():
        x = h_ref[...] + agg_ref[0] + agg_ref[1]
        t = jnp.maximum(
            jnp.dot(x.astype(jnp.bfloat16), w1_ref[...].astype(jnp.bfloat16),
                    preferred_element_type=jnp.float32) + b1_ref[...], 0.0)
        o = jnp.dot(t.astype(jnp.bfloat16), w2_ref[...].astype(jnp.bfloat16),
                    preferred_element_type=jnp.float32) + b2_ref[...]
        o_scr[pl.ds(i * _BLK, _BLK), :] = o
        st = jnp.concatenate([jnp.sum(o, 0, keepdims=True),
                              jnp.sum(o * o, 0, keepdims=True)], axis=0)

        @pl.when(i == 0)
        def _():
            sums_scr[...] = st

        @pl.when(i > 0)
        def _():
            sums_scr[...] += st

    @pl.when(i >= _GRID)
    def _():
        j = i - _GRID
        inv_n = 1.0 / N
        mean = sums_scr[pl.ds(0, 1), :] * inv_n
        ex2 = sums_scr[pl.ds(1, 1), :] * inv_n
        var = ex2 - mean * mean
        scale = g_ref[...] * lax.rsqrt(var + BN_EPS)
        o = o_scr[pl.ds(j * _BLK, _BLK), :]
        out_ref[...] = jnp.maximum((o - mean) * scale + bt_ref[...], 0.0)


_mlpbn_call = pl.pallas_call(
    _mlpbn_body,
    grid=(2 * _GRID,),
    in_specs=[
        pl.BlockSpec((_BLK, D), lambda i: (jnp.minimum(i, _GRID - 1), 0)),
        pl.BlockSpec((2, _BLK, D), lambda i: (0, jnp.minimum(i, _GRID - 1), 0)),
        pl.BlockSpec((D, D), lambda i: (0, 0)),
        pl.BlockSpec((1, D), lambda i: (0, 0)),
        pl.BlockSpec((D, D), lambda i: (0, 0)),
        pl.BlockSpec((1, D), lambda i: (0, 0)),
        pl.BlockSpec((1, D), lambda i: (0, 0)),
        pl.BlockSpec((1, D), lambda i: (0, 0)),
    ],
    out_specs=pl.BlockSpec((_BLK, D), lambda i: (jnp.maximum(i - _GRID, 0), 0)),
    out_shape=jax.ShapeDtypeStruct((N, D), jnp.float32),
    scratch_shapes=[
        pltpu.VMEM((N, D), jnp.float32),
        pltpu.VMEM((2, D), jnp.float32),
    ],
)


def kernel(x, edge_index,
           W1_0, b1_0, W2_0, b2_0, gamma_0, beta_0,
           W1_1, b1_1, W2_1, b2_1, gamma_1, beta_1,
           W1_2, b1_2, W2_2, b2_2, gamma_2, beta_2):
    npad_e = EPP - EPW
    pad_src = (jnp.arange(NW * npad_e, dtype=jnp.int32) % N).reshape(NW, npad_e)
    pad_dst = N + (jnp.arange(NW * npad_e, dtype=jnp.int32)
                   % (NPAD - N)).reshape(NW, npad_e)
    src = jnp.concatenate([edge_index[0].reshape(NW, EPW), pad_src],
                          axis=1).reshape(-1)
    dst = jnp.concatenate([edge_index[1].reshape(NW, EPW), pad_dst],
                          axis=1).reshape(-1)
    layer_params = [
        (W1_0, b1_0, W2_0, b2_0, gamma_0, beta_0),
        (W1_1, b1_1, W2_1, b2_1, gamma_1, beta_1),
        (W1_2, b1_2, W2_2, b2_2, gamma_2, beta_2),
    ]
    h = x
    outs = []
    for (W1, b1, W2, b2, g, bt) in layer_params:
        aggf = _get_agg_call()(h, src, dst)
        agg2 = aggf.reshape(2, NPAD, D)
        h = _mlpbn_call(h, agg2, W1, b1.reshape(1, D), W2, b2.reshape(1, D),
                        g.reshape(1, D), bt.reshape(1, D))
        outs.append(h)
    return tuple(outs)


# speedup vs baseline: 12.1464x; 1.0288x over previous
import functools

import jax
import jax.numpy as jnp
from jax import lax
from jax.experimental import pallas as pl
from jax.experimental.pallas import tpu as pltpu
from jax.experimental.pallas import tpu_sc as plsc

N = 10000
E = 320000
D = 128
BN_EPS = 1e-5

NC = 2
NS = 16
NW = NC * NS
EPW = E // NW
C = 48
EPP = 10080
NCHUNK = EPP // C
NPAD = 10240
RPT = NPAD // NS


G = 5


def _agg_body(h_hbm, src_hbm, dst_hbm, out_hbm, src_i, dst_i, ring_v, agg_sh,
              *sems):
    sem_i = sems[0:G]
    sem_g = sems[G:2 * G]
    sem_s = sems[2 * G:3 * G]
    c = lax.axis_index("c")
    s = lax.axis_index("s")
    w = c * NS + s
    base_e = w * EPP

    def zbody(i, _):
        r = i // 8
        col = (i % 8) * 16
        ring_v[0, r, pl.ds(col, 16)] = jnp.zeros((16,), jnp.float32)
        return 0
    lax.fori_loop(0, C * 8, zbody, 0)
    base_r = s * RPT

    ZC = 32

    def zcopy(j, _):
        pltpu.sync_copy(ring_v.at[0, pl.ds(0, ZC)],
                        agg_sh.at[pl.ds(base_r + j * ZC, ZC)])
        return 0
    lax.fori_loop(0, RPT // ZC, zcopy, 0)
    plsc.subcore_barrier()

    def idx_issue(t, b):
        off = base_e + t * C
        pltpu.async_copy(src_hbm.at[pl.ds(off, C)], src_i.at[b], sem_i[b])
        pltpu.async_copy(dst_hbm.at[pl.ds(off, C)], dst_i.at[b], sem_i[b])

    def idx_wait(t, b):
        off = base_e + t * C
        pltpu.make_async_copy(src_hbm.at[pl.ds(off, C)], src_i.at[b], sem_i[b]).wait()
        pltpu.make_async_copy(dst_hbm.at[pl.ds(off, C)], dst_i.at[b], sem_i[b]).wait()

    def gather_issue(b):
        pltpu.async_copy(h_hbm.at[src_i.at[b]], ring_v.at[b], sem_g[b])

    def gather_wait(b):
        pltpu.make_async_copy(h_hbm.at[src_i.at[b]], ring_v.at[b], sem_g[b]).wait()

    def scat_issue(b):
        pltpu.async_copy(ring_v.at[b], agg_sh.at[dst_i.at[b]], sem_s[b], add=True)

    def scat_wait(b):
        pltpu.make_async_copy(ring_v.at[b], agg_sh.at[dst_i.at[b]], sem_s[b]).wait()

    for t in range(4):
        idx_issue(t, t)
    for t in range(2):
        idx_wait(t, t)
        gather_issue(t)

    def body(j, _):
        for b in range(G):
            t = j * G + b

            @pl.when(t + 4 < NCHUNK)
            def _():
                idx_issue(t + 4, (b + 4) % G)

            @pl.when(t >= 3)
            def _():
                scat_wait((b + 2) % G)

            @pl.when(t + 2 < NCHUNK)
            def _():
                idx_wait(t + 2, (b + 2) % G)
                gather_issue((b + 2) % G)
            gather_wait(b)
            scat_issue(b)
        return 0
    lax.fori_loop(0, NCHUNK // G, body, 0)
    scat_wait((NCHUNK - 3) % G)
    scat_wait((NCHUNK - 2) % G)
    scat_wait((NCHUNK - 1) % G)
    plsc.subcore_barrier()

    out_base = c * NPAD + s * RPT
    pltpu.sync_copy(agg_sh.at[pl.ds(base_r, RPT)], out_hbm.at[pl.ds(out_base, RPT)])


_agg_call_cache = []


def _get_agg_call():
    if not _agg_call_cache:
        _agg_call_cache.append(functools.partial(
            pl.kernel,
            mesh=plsc.VectorSubcoreMesh(core_axis_name="c", subcore_axis_name="s"),
            out_type=jax.ShapeDtypeStruct((2 * NPAD, D), jnp.float32),
            scratch_types=[
                pltpu.VMEM((G, C), jnp.int32),
                pltpu.VMEM((G, C), jnp.int32),
                pltpu.VMEM((G, C, D), jnp.float32),
                pltpu.VMEM_SHARED((NPAD, D), jnp.float32),
            ] + [pltpu.SemaphoreType.DMA] * (3 * G),
        )(_agg_body))
    return _agg_call_cache[0]


_BLK = 2000
_GRID = N // _BLK


def _mlpbn_body(h_ref, agg_ref, w1_ref, b1_ref, w2_ref, b2_ref, g_ref, bt_ref,
                out_ref, o_scr, sums_scr):
    i = pl.program_id(0)

    @pl.when(i < _GRID)
    def _():
        x = h_ref[...] + agg_ref[0] + agg_ref[1]
        t = jnp.maximum(
            jnp.dot(x.astype(jnp.bfloat16), w1_ref[...].astype(jnp.bfloat16),
                    preferred_element_type=jnp.float32) + b1_ref[...], 0.0)
        o = jnp.dot(t.astype(jnp.bfloat16), w2_ref[...].astype(jnp.bfloat16),
                    preferred_element_type=jnp.float32) + b2_ref[...]
        o_scr[pl.ds(i * _BLK, _BLK), :] = o
        st = jnp.concatenate([jnp.sum(o, 0, keepdims=True),
                              jnp.sum(o * o, 0, keepdims=True)], axis=0)

        @pl.when(i == 0)
        def _():
            sums_scr[...] = st

        @pl.when(i > 0)
        def _():
            sums_scr[...] += st

    @pl.when(i >= _GRID)
    def _():
        j = i - _GRID
        inv_n = 1.0 / N
        mean = sums_scr[pl.ds(0, 1), :] * inv_n
        ex2 = sums_scr[pl.ds(1, 1), :] * inv_n
        var = ex2 - mean * mean
        scale = g_ref[...] * lax.rsqrt(var + BN_EPS)
        o = o_scr[pl.ds(j * _BLK, _BLK), :]
        out_ref[...] = jnp.maximum((o - mean) * scale + bt_ref[...], 0.0)


_mlpbn_call = pl.pallas_call(
    _mlpbn_body,
    grid=(2 * _GRID,),
    in_specs=[
        pl.BlockSpec((_BLK, D), lambda i: (jnp.minimum(i, _GRID - 1), 0)),
        pl.BlockSpec((2, _BLK, D), lambda i: (0, jnp.minimum(i, _GRID - 1), 0)),
        pl.BlockSpec((D, D), lambda i: (0, 0)),
        pl.BlockSpec((1, D), lambda i: (0, 0)),
        pl.BlockSpec((D, D), lambda i: (0, 0)),
        pl.BlockSpec((1, D), lambda i: (0, 0)),
        pl.BlockSpec((1, D), lambda i: (0, 0)),
        pl.BlockSpec((1, D), lambda i: (0, 0)),
    ],
    out_specs=pl.BlockSpec((_BLK, D), lambda i: (jnp.maximum(i - _GRID, 0), 0)),
    out_shape=jax.ShapeDtypeStruct((N, D), jnp.float32),
    scratch_shapes=[
        pltpu.VMEM((N, D), jnp.float32),
        pltpu.VMEM((2, D), jnp.float32),
    ],
)


def kernel(x, edge_index,
           W1_0, b1_0, W2_0, b2_0, gamma_0, beta_0,
           W1_1, b1_1, W2_1, b2_1, gamma_1, beta_1,
           W1_2, b1_2, W2_2, b2_2, gamma_2, beta_2):
    npad_e = EPP - EPW
    pad_src = (jnp.arange(NW * npad_e, dtype=jnp.int32) % N).reshape(NW, npad_e)
    pad_dst = N + (jnp.arange(NW * npad_e, dtype=jnp.int32)
                   % (NPAD - N)).reshape(NW, npad_e)
    src = jnp.concatenate([edge_index[0].reshape(NW, EPW), pad_src],
                          axis=1).reshape(-1)
    dst = jnp.concatenate([edge_index[1].reshape(NW, EPW), pad_dst],
                          axis=1).reshape(-1)
    layer_params = [
        (W1_0, b1_0, W2_0, b2_0, gamma_0, beta_0),
        (W1_1, b1_1, W2_1, b2_1, gamma_1, beta_1),
        (W1_2, b1_2, W2_2, b2_2, gamma_2, beta_2),
    ]
    h = x
    outs = []
    for (W1, b1, W2, b2, g, bt) in layer_params:
        aggf = _get_agg_call()(h, src, dst)
        agg2 = aggf.reshape(2, NPAD, D)
        h = _mlpbn_call(h, agg2, W1, b1.reshape(1, D), W2, b2.reshape(1, D),
                        g.reshape(1, D), bt.reshape(1, D))
        outs.append(h)
    return tuple(outs)


# speedup vs baseline: 12.2887x; 1.0117x over previous
import functools

import jax
import jax.numpy as jnp
from jax import lax
from jax.experimental import pallas as pl
from jax.experimental.pallas import tpu as pltpu
from jax.experimental.pallas import tpu_sc as plsc

N = 10000
E = 320000
D = 128
BN_EPS = 1e-5

NC = 2
NS = 16
NW = NC * NS
EPW = E // NW
C = 48
EPP = 10080
NCHUNK = EPP // C
NPAD = 10240
RPT = NPAD // NS


G = 5


def _agg_body(h_hbm, src_hbm, dst_hbm, out_hbm, src_i, dst_i, ring_v, agg_sh,
              *sems):
    sem_i = sems[0:G]
    sem_g = sems[G:2 * G]
    sem_s = sems[2 * G:3 * G]
    c = lax.axis_index("c")
    s = lax.axis_index("s")
    w = c * NS + s
    base_e = w * EPP

    def zbody(i, _):
        r = i // 8
        col = (i % 8) * 16
        ring_v[0, r, pl.ds(col, 16)] = jnp.zeros((16,), jnp.float32)
        return 0
    lax.fori_loop(0, C * 8, zbody, 0)
    base_r = s * RPT

    ZC = 32
    sem_z = sems[3 * G]

    def zissue(j, _):
        pltpu.async_copy(ring_v.at[0, pl.ds(0, ZC)],
                         agg_sh.at[pl.ds(base_r + j * ZC, ZC)], sem_z)
        return 0
    lax.fori_loop(0, RPT // ZC, zissue, 0)

    def zdrain(j, _):
        pltpu.make_async_copy(ring_v.at[0, pl.ds(0, ZC)],
                              agg_sh.at[pl.ds(base_r + j * ZC, ZC)], sem_z).wait()
        return 0

    def idx_issue(t, b):
        off = base_e + t * C
        pltpu.async_copy(src_hbm.at[pl.ds(off, C)], src_i.at[b], sem_i[b])
        pltpu.async_copy(dst_hbm.at[pl.ds(off, C)], dst_i.at[b], sem_i[b])

    def idx_wait(t, b):
        off = base_e + t * C
        pltpu.make_async_copy(src_hbm.at[pl.ds(off, C)], src_i.at[b], sem_i[b]).wait()
        pltpu.make_async_copy(dst_hbm.at[pl.ds(off, C)], dst_i.at[b], sem_i[b]).wait()

    def gather_issue(b):
        pltpu.async_copy(h_hbm.at[src_i.at[b]], ring_v.at[b], sem_g[b])

    def gather_wait(b):
        pltpu.make_async_copy(h_hbm.at[src_i.at[b]], ring_v.at[b], sem_g[b]).wait()

    def scat_issue(b):
        pltpu.async_copy(ring_v.at[b], agg_sh.at[dst_i.at[b]], sem_s[b], add=True)

    def scat_wait(b):
        pltpu.make_async_copy(ring_v.at[b], agg_sh.at[dst_i.at[b]], sem_s[b]).wait()

    for t in range(4):
        idx_issue(t, t)
    lax.fori_loop(0, RPT // ZC, zdrain, 0)
    for t in range(2):
        idx_wait(t, t)
        gather_issue(t)
    plsc.subcore_barrier()

    def body(j, _):
        for b in range(G):
            t = j * G + b

            @pl.when(t + 4 < NCHUNK)
            def _():
                idx_issue(t + 4, (b + 4) % G)

            @pl.when(t >= 3)
            def _():
                scat_wait((b + 2) % G)

            @pl.when(t + 2 < NCHUNK)
            def _():
                idx_wait(t + 2, (b + 2) % G)
                gather_issue((b + 2) % G)
            gather_wait(b)
            scat_issue(b)
        return 0
    lax.fori_loop(0, NCHUNK // G, body, 0)
    scat_wait((NCHUNK - 3) % G)
    scat_wait((NCHUNK - 2) % G)
    scat_wait((NCHUNK - 1) % G)
    plsc.subcore_barrier()

    out_base = c * NPAD + s * RPT
    pltpu.sync_copy(agg_sh.at[pl.ds(base_r, RPT)], out_hbm.at[pl.ds(out_base, RPT)])


_agg_call_cache = []


def _get_agg_call():
    if not _agg_call_cache:
        _agg_call_cache.append(functools.partial(
            pl.kernel,
            mesh=plsc.VectorSubcoreMesh(core_axis_name="c", subcore_axis_name="s"),
            out_type=jax.ShapeDtypeStruct((2 * NPAD, D), jnp.float32),
            scratch_types=[
                pltpu.VMEM((G, C), jnp.int32),
                pltpu.VMEM((G, C), jnp.int32),
                pltpu.VMEM((G, C, D), jnp.float32),
                pltpu.VMEM_SHARED((NPAD, D), jnp.float32),
            ] + [pltpu.SemaphoreType.DMA] * (3 * G + 1),
        )(_agg_body))
    return _agg_call_cache[0]


_BLK = 2000
_GRID = N // _BLK


def _mlpbn_body(h_ref, agg_ref, w1_ref, b1_ref, w2_ref, b2_ref, g_ref, bt_ref,
                out_ref, o_scr, sums_scr):
    i = pl.program_id(0)

    @pl.when(i < _GRID)
    def _():
        x = h_ref[...] + agg_ref[0] + agg_ref[1]
        t = jnp.maximum(
            jnp.dot(x.astype(jnp.bfloat16), w1_ref[...].astype(jnp.bfloat16),
                    preferred_element_type=jnp.float32) + b1_ref[...], 0.0)
        o = jnp.dot(t.astype(jnp.bfloat16), w2_ref[...].astype(jnp.bfloat16),
                    preferred_element_type=jnp.float32) + b2_ref[...]
        o_scr[pl.ds(i * _BLK, _BLK), :] = o
        st = jnp.concatenate([jnp.sum(o, 0, keepdims=True),
                              jnp.sum(o * o, 0, keepdims=True)], axis=0)

        @pl.when(i == 0)
        def _():
            sums_scr[...] = st

        @pl.when(i > 0)
        def _():
            sums_scr[...] += st

    @pl.when(i >= _GRID)
    def _():
        j = i - _GRID
        inv_n = 1.0 / N
        mean = sums_scr[pl.ds(0, 1), :] * inv_n
        ex2 = sums_scr[pl.ds(1, 1), :] * inv_n
        var = ex2 - mean * mean
        scale = g_ref[...] * lax.rsqrt(var + BN_EPS)
        o = o_scr[pl.ds(j * _BLK, _BLK), :]
        out_ref[...] = jnp.maximum((o - mean) * scale + bt_ref[...], 0.0)


_mlpbn_call = pl.pallas_call(
    _mlpbn_body,
    grid=(2 * _GRID,),
    in_specs=[
        pl.BlockSpec((_BLK, D), lambda i: (jnp.minimum(i, _GRID - 1), 0)),
        pl.BlockSpec((2, _BLK, D), lambda i: (0, jnp.minimum(i, _GRID - 1), 0)),
        pl.BlockSpec((D, D), lambda i: (0, 0)),
        pl.BlockSpec((1, D), lambda i: (0, 0)),
        pl.BlockSpec((D, D), lambda i: (0, 0)),
        pl.BlockSpec((1, D), lambda i: (0, 0)),
        pl.BlockSpec((1, D), lambda i: (0, 0)),
        pl.BlockSpec((1, D), lambda i: (0, 0)),
    ],
    out_specs=pl.BlockSpec((_BLK, D), lambda i: (jnp.maximum(i - _GRID, 0), 0)),
    out_shape=jax.ShapeDtypeStruct((N, D), jnp.float32),
    scratch_shapes=[
        pltpu.VMEM((N, D), jnp.float32),
        pltpu.VMEM((2, D), jnp.float32),
    ],
)


def kernel(x, edge_index,
           W1_0, b1_0, W2_0, b2_0, gamma_0, beta_0,
           W1_1, b1_1, W2_1, b2_1, gamma_1, beta_1,
           W1_2, b1_2, W2_2, b2_2, gamma_2, beta_2):
    npad_e = EPP - EPW
    pad_src = (jnp.arange(NW * npad_e, dtype=jnp.int32) % N).reshape(NW, npad_e)
    pad_dst = N + (jnp.arange(NW * npad_e, dtype=jnp.int32)
                   % (NPAD - N)).reshape(NW, npad_e)
    src = jnp.concatenate([edge_index[0].reshape(NW, EPW), pad_src],
                          axis=1).reshape(-1)
    dst = jnp.concatenate([edge_index[1].reshape(NW, EPW), pad_dst],
                          axis=1).reshape(-1)
    layer_params = [
        (W1_0, b1_0, W2_0, b2_0, gamma_0, beta_0),
        (W1_1, b1_1, W2_1, b2_1, gamma_1, beta_1),
        (W1_2, b1_2, W2_2, b2_2, gamma_2, beta_2),
    ]
    h = x
    outs = []
    for (W1, b1, W2, b2, g, bt) in layer_params:
        aggf = _get_agg_call()(h, src, dst)
        agg2 = aggf.reshape(2, NPAD, D)
        h = _mlpbn_call(h, agg2, W1, b1.reshape(1, D), W2, b2.reshape(1, D),
                        g.reshape(1, D), bt.reshape(1, D))
        outs.append(h)
    return tuple(outs)
